# baseline (speedup 1.0000x reference)
import functools

import jax
import jax.numpy as jnp
from jax import lax
from jax.experimental import pallas as pl
from jax.experimental.pallas import tpu as pltpu
from jax.experimental.pallas import tpu_sc as plsc

B = 16384
F = 26
FS = 32
D = 16
NUM_FIELD_ROWS = 100000
TOTAL_ROWS = F * NUM_FIELD_ROWS

NC, NS = 2, 16
NW = NC * NS
ROWS_PER_W = B // NW
IDX_PER_W = ROWS_PER_W * FS
GROUP = 8
NGROUP = ROWS_PER_W // GROUP


def _sc_body(xi_hbm, emb_hbm, lint_hbm, cross_hbm, lin_hbm,
             idx_v, ebuf, lbuf, cacc, lacc, sem0):
    wid = lax.axis_index("s") * NC + lax.axis_index("c")
    base_idx = wid * IDX_PER_W
    pltpu.sync_copy(xi_hbm.at[pl.ds(base_idx, IDX_PER_W)], idx_v)

    mask7 = jnp.full((16,), 7, jnp.int32)
    maskt = jnp.full((16,), ~7, jnp.int32)
    lane = lax.iota(jnp.int32, 16)

    def fire(c):
        off = pl.multiple_of(c * FS, 8)
        handles = []
        for v in range(2):
            ivec = idx_v[pl.ds(off + v * 16, 16)]
            base = ivec & maskt
            for l in range(min(16, F - v * 16)):
                j = v * 16 + l
                r8 = pl.multiple_of(base[l], 8)
                handles.append(pltpu.async_copy(
                    emb_hbm.at[pl.ds(r8, 8), :], ebuf.at[j], sem0))
                handles.append(pltpu.async_copy(
                    lint_hbm.at[pl.ds(r8, 8), :], lbuf.at[j], sem0))
        return handles

    def compute(c, h):
        off = pl.multiple_of(c * FS, 8)
        s = jnp.zeros((D,), jnp.float32)
        q = jnp.zeros((D,), jnp.float32)
        subs = []
        for v in range(2):
            ivec = idx_v[pl.ds(off + v * 16, 16)]
            svec = ivec & mask7
            for l in range(min(16, F - v * 16)):
                subs.append(svec[l])
        for j, sub in enumerate(subs):
            v = ebuf[j, sub, :]
            s = s + v
            q = q + v * v
        cacc[h] = 0.5 * (s * s - q)
        raw0 = idx_v[pl.ds(off, 16)]
        raw1 = idx_v[pl.ds(off + 16, 16)]
        s0 = raw0 & mask7
        s1 = raw1 & mask7
        zero16 = jnp.zeros((16,), jnp.int32)
        j1 = jnp.minimum(lane + 16, F - 1)
        v0 = plsc.load_gather(lbuf, [lane, s0, zero16])
        v1 = plsc.load_gather(lbuf, [j1, s1, zero16])
        tvec = v0 + jnp.where(lane + 16 < F, v1, 0.0)
        t = jnp.sum(tvec, axis=0)
        plsc.store_scatter(lacc, [jnp.full((16,), h, jnp.int32)],
                           jnp.zeros((16,), jnp.float32) + t, mask=lane < 1)

    base_row = wid * ROWS_PER_W

    @pl.loop(0, NGROUP)
    def _group(g):
        for h in range(GROUP):
            c = g * GROUP + h
            for hd in fire(c):
                hd.wait()
            compute(c, h)
        row8 = pl.multiple_of(base_row + g * GROUP, 8)
        pltpu.sync_copy(cacc, cross_hbm.at[pl.ds(row8, GROUP), :])
        pltpu.sync_copy(lacc, lin_hbm.at[pl.ds(row8, GROUP)])


@functools.partial(jax.jit, static_argnames=("interpret",))
def _sc_gather_fm(xi, emb_table, lin_table, interpret=False):
    mesh = plsc.VectorSubcoreMesh(core_axis_name="c", subcore_axis_name="s",
                                  num_cores=NC, num_subcores=NS)
    return pl.kernel(
        _sc_body,
        out_type=(
            jax.ShapeDtypeStruct((B, D), jnp.float32),
            jax.ShapeDtypeStruct((B,), jnp.float32),
        ),
        mesh=mesh,
        compiler_params=pltpu.CompilerParams(needs_layout_passes=False,
                                             use_tc_tiling_on_sc=True),
        scratch_types=[
            pltpu.VMEM((IDX_PER_W,), jnp.int32),
            pltpu.VMEM((F, 8, D), jnp.float32),
            pltpu.VMEM((F, 8, 1), jnp.float32),
            pltpu.VMEM((GROUP, D), jnp.float32),
            pltpu.VMEM((GROUP,), jnp.float32),
            pltpu.SemaphoreType.DMA,
        ],
        interpret=interpret,
    )(xi, emb_table, lin_table)


def _bn(v, g, b, eps=1e-5):
    m = jnp.mean(v, axis=0, keepdims=True)
    var = jnp.mean((v - m) ** 2, axis=0, keepdims=True)
    return (v - m) * lax.rsqrt(var + eps) * g + b


def _tc_body(cross_ref, lin_ref, linb_ref, g0_ref, b0_ref, W1_ref, b1_ref,
             g1_ref, bb1_ref, W2_ref, b2_ref, g2_ref, bb2_ref, W3_ref,
             b3_ref, out_ref):
    h = _bn(cross_ref[...], g0_ref[...], b0_ref[...])
    h = jnp.dot(h, W1_ref[...], preferred_element_type=jnp.float32) + b1_ref[...]
    h = jnp.maximum(_bn(h, g1_ref[...], bb1_ref[...]), 0.0)
    h = jnp.dot(h, W2_ref[...], preferred_element_type=jnp.float32) + b2_ref[...]
    h = jnp.maximum(_bn(h, g2_ref[...], bb2_ref[...]), 0.0)
    o = jnp.dot(h, W3_ref[...], preferred_element_type=jnp.float32) + b3_ref[...]
    z = lin_ref[...] + linb_ref[...] + o
    out_ref[...] = 1.0 / (1.0 + jnp.exp(-z))


@functools.partial(jax.jit, static_argnames=("interpret",))
def _tc_mlp(cross, lin, lin_bias, bn0_g, bn0_b, W1, b1, bn1_g, bn1_b,
            W2, b2, bn2_g, bn2_b, W3, b3, interpret=False):
    args = (
        cross,
        lin.reshape(B, 1),
        lin_bias.reshape(1, 1),
        bn0_g.reshape(1, D), bn0_b.reshape(1, D),
        W1, b1.reshape(1, -1), bn1_g.reshape(1, -1), bn1_b.reshape(1, -1),
        W2, b2.reshape(1, -1), bn2_g.reshape(1, -1), bn2_b.reshape(1, -1),
        W3, b3.reshape(1, 1),
    )
    out = pl.pallas_call(
        _tc_body,
        out_shape=jax.ShapeDtypeStruct((B, 1), jnp.float32),
        interpret=interpret,
    )(*args)
    return out.reshape(B)


def kernel(x, emb_table, lin_table, lin_bias, bn0_g, bn0_b, W1, b1,
           bn1_g, bn1_b, W2, b2, bn2_g, bn2_b, W3, b3):
    offsets = (jnp.arange(F, dtype=x.dtype) * NUM_FIELD_ROWS)[None, :]
    xi = x + offsets
    xi = jnp.pad(xi, ((0, 0), (0, FS - F)), mode="edge").reshape(-1)
    cross, lin = _sc_gather_fm(xi, emb_table, lin_table)
    return _tc_mlp(cross, lin, lin_bias, bn0_g, bn0_b, W1, b1, bn1_g, bn1_b,
                   W2, b2, bn2_g, bn2_b, W3, b3)


# speedup vs baseline: 2.1514x; 2.1514x over previous
import functools

import jax
import jax.numpy as jnp
from jax import lax
from jax.experimental import pallas as pl
from jax.experimental.pallas import tpu as pltpu
from jax.experimental.pallas import tpu_sc as plsc

B = 16384
F = 26
D = 16
NUM_FIELD_ROWS = 100000
TOTAL_ROWS = F * NUM_FIELD_ROWS

NC, NS = 2, 16
NW = NC * NS
ROWS_PER_W = B // NW
IDX_PER_W = ROWS_PER_W * F
CHUNK_ROWS = 4
CHUNK_IDX = CHUNK_ROWS * F
NCHUNK = ROWS_PER_W // CHUNK_ROWS


def _sc_emb_body(xi_hbm, emb_hbm, cross_hbm, idx_v, ebuf, cacc, sem0):
    wid = lax.axis_index("s") * NC + lax.axis_index("c")
    base_idx = wid * IDX_PER_W
    pltpu.sync_copy(xi_hbm.at[pl.ds(base_idx, IDX_PER_W)], idx_v)

    @pl.loop(0, NCHUNK)
    def _chunk(c):
        off = pl.multiple_of(c * CHUNK_IDX, 8)
        cp = pltpu.async_copy(emb_hbm.at[idx_v.at[pl.ds(off, CHUNK_IDX)]],
                              ebuf, sem0)
        cp.wait()
        for r in range(CHUNK_ROWS):
            s = jnp.zeros((D,), jnp.float32)
            q = jnp.zeros((D,), jnp.float32)
            for j in range(F):
                v = ebuf[r * F + j]
                s = s + v
                q = q + v * v
            cacc[c * CHUNK_ROWS + r] = 0.5 * (s * s - q)

    base_row = wid * ROWS_PER_W
    pltpu.sync_copy(cacc, cross_hbm.at[pl.ds(base_row, ROWS_PER_W)])


def _sc_lin_body(xi_hbm, lint_hbm, lin_hbm, idx_v, lbuf, lacc, sem0):
    wid = lax.axis_index("s") * NC + lax.axis_index("c")
    base_idx = wid * IDX_PER_W
    pltpu.sync_copy(xi_hbm.at[pl.ds(base_idx, IDX_PER_W)], idx_v)
    lane = lax.iota(jnp.int32, 16)

    @pl.loop(0, NCHUNK)
    def _chunk(c):
        off = pl.multiple_of(c * CHUNK_IDX, 8)
        cp = pltpu.async_copy(lint_hbm.at[idx_v.at[pl.ds(off, CHUNK_IDX)]],
                              lbuf, sem0)
        cp.wait()
        t = jnp.zeros((16,), jnp.float32)
        for j in range(F):
            idx = jnp.minimum(lane * F + j, CHUNK_IDX - 1)
            t = t + plsc.load_gather(lbuf, [idx])
        rowi = jnp.minimum(c * CHUNK_ROWS + lane, ROWS_PER_W - 1)
        plsc.store_scatter(lacc, [rowi], t, mask=lane < CHUNK_ROWS)

    base_row = wid * ROWS_PER_W
    pltpu.sync_copy(lacc, lin_hbm.at[pl.ds(base_row, ROWS_PER_W)])


@functools.partial(jax.jit, static_argnames=("interpret",))
def _sc_gather_fm(xi, emb_table, lin_flat, interpret=False):
    mesh = plsc.VectorSubcoreMesh(core_axis_name="c", subcore_axis_name="s",
                                  num_cores=NC, num_subcores=NS)
    params = pltpu.CompilerParams(needs_layout_passes=False,
                                  use_tc_tiling_on_sc=False)
    cross = pl.kernel(
        _sc_emb_body,
        out_type=jax.ShapeDtypeStruct((B, D), jnp.float32),
        mesh=mesh,
        compiler_params=params,
        scratch_types=[
            pltpu.VMEM((IDX_PER_W,), jnp.int32),
            pltpu.VMEM((CHUNK_IDX, D), jnp.float32),
            pltpu.VMEM((ROWS_PER_W, D), jnp.float32),
            pltpu.SemaphoreType.DMA,
        ],
        interpret=interpret,
    )(xi, emb_table)
    lin = pl.kernel(
        _sc_lin_body,
        out_type=jax.ShapeDtypeStruct((B,), jnp.float32),
        mesh=mesh,
        compiler_params=params,
        scratch_types=[
            pltpu.VMEM((IDX_PER_W,), jnp.int32),
            pltpu.VMEM((CHUNK_IDX,), jnp.float32),
            pltpu.VMEM((ROWS_PER_W,), jnp.float32),
            pltpu.SemaphoreType.DMA,
        ],
        interpret=interpret,
    )(xi, lin_flat)
    return cross, lin


def _bn(v, g, b, eps=1e-5):
    m = jnp.mean(v, axis=0, keepdims=True)
    var = jnp.mean((v - m) ** 2, axis=0, keepdims=True)
    return (v - m) * lax.rsqrt(var + eps) * g + b


def _tc_body(cross_ref, lin_ref, linb_ref, g0_ref, b0_ref, W1_ref, b1_ref,
             g1_ref, bb1_ref, W2_ref, b2_ref, g2_ref, bb2_ref, W3_ref,
             b3_ref, out_ref):
    h = _bn(cross_ref[...], g0_ref[...], b0_ref[...])
    h = jnp.dot(h, W1_ref[...], preferred_element_type=jnp.float32) + b1_ref[...]
    h = jnp.maximum(_bn(h, g1_ref[...], bb1_ref[...]), 0.0)
    h = jnp.dot(h, W2_ref[...], preferred_element_type=jnp.float32) + b2_ref[...]
    h = jnp.maximum(_bn(h, g2_ref[...], bb2_ref[...]), 0.0)
    o = jnp.dot(h, W3_ref[...], preferred_element_type=jnp.float32) + b3_ref[...]
    z = lin_ref[...] + linb_ref[...] + o
    out_ref[...] = 1.0 / (1.0 + jnp.exp(-z))


@functools.partial(jax.jit, static_argnames=("interpret",))
def _tc_mlp(cross, lin, lin_bias, bn0_g, bn0_b, W1, b1, bn1_g, bn1_b,
            W2, b2, bn2_g, bn2_b, W3, b3, interpret=False):
    args = (
        cross,
        lin.reshape(B, 1),
        lin_bias.reshape(1, 1),
        bn0_g.reshape(1, D), bn0_b.reshape(1, D),
        W1, b1.reshape(1, -1), bn1_g.reshape(1, -1), bn1_b.reshape(1, -1),
        W2, b2.reshape(1, -1), bn2_g.reshape(1, -1), bn2_b.reshape(1, -1),
        W3, b3.reshape(1, 1),
    )
    out = pl.pallas_call(
        _tc_body,
        out_shape=jax.ShapeDtypeStruct((B, 1), jnp.float32),
        interpret=interpret,
    )(*args)
    return out.reshape(B)


def kernel(x, emb_table, lin_table, lin_bias, bn0_g, bn0_b, W1, b1,
           bn1_g, bn1_b, W2, b2, bn2_g, bn2_b, W3, b3):
    offsets = (jnp.arange(F, dtype=x.dtype) * NUM_FIELD_ROWS)[None, :]
    xi = (x + offsets).reshape(-1)
    cross, lin = _sc_gather_fm(xi, emb_table, lin_table.reshape(-1))
    return _tc_mlp(cross, lin, lin_bias, bn0_g, bn0_b, W1, b1, bn1_g, bn1_b,
                   W2, b2, bn2_g, bn2_b, W3, b3)
